# baseline (device time: 217587 ns/iter reference)
import jax
import jax.numpy as jnp
from jax import lax
from jax.experimental import pallas as pl
from jax.experimental.pallas import tpu as pltpu

N_DEV = 8
SQ = 2048
SKV = 2048
HQ = 8
DH = 128
DM = 1024
BLK = 64
CHUNK = SQ // N_DEV
LOG2E = 1.4426950408889634
SCALE = 0.08838834764831843 * LOG2E
NEG = -1.5e9


def kernel(x, Wq, K_ext, V_ext, Wo):
    def body(x_ref, wq_ref, kt_ref, vt_ref, wo_ref, out_ref,
             q_buf, ctx_buf, ml_buf, recv_ctx, recv_ml,
             rs_send, rs_recv, ml_send, ml_recv, ag_send, ag_recv,
             agl_send, agl_recv):
        my = lax.axis_index("i")
        left = lax.rem(my + N_DEV - 1, N_DEV)
        right = lax.rem(my + 1, N_DEV)

        barrier = pltpu.get_barrier_semaphore()
        for nbr in (left, right):
            pl.semaphore_signal(barrier, inc=1, device_id=(nbr,),
                                device_id_type=pl.DeviceIdType.MESH)
        pl.semaphore_wait(barrier, 2)

        def qproj_step(c, carry):
            r0 = c * CHUNK
            q_buf[pl.ds(r0, CHUNK), :] = (lax.dot_general(
                x_ref[0, pl.ds(r0, CHUNK), :], wq_ref[...],
                (((1,), (0,)), ((), ())),
                preferred_element_type=jnp.float32) * SCALE).astype(jnp.bfloat16)
            return carry

        lax.fori_loop(0, N_DEV, qproj_step, 0)

        kb_off = my * (SKV // BLK)

        def compute_chunk(r0):
            qb = (lax.broadcasted_iota(jnp.int32, (CHUNK, SKV), 0) + r0) // BLK
            kb = lax.broadcasted_iota(jnp.int32, (CHUNK, SKV), 1) // BLK + kb_off
            active = (qb == kb) | (kb == 0) | (lax.rem(qb + kb, 3) == 0)
            bias = jnp.where(active, 0.0, NEG).astype(jnp.float32)
            for h in range(HQ):
                qh = q_buf[pl.ds(r0, CHUNK), h * DH:(h + 1) * DH]
                s = lax.dot_general(qh, kt_ref[h], (((1,), (1,)), ((), ())),
                                    preferred_element_type=jnp.float32) + bias
                m = jnp.max(s, axis=1, keepdims=True)
                p = jnp.exp2(s - m)
                lsum = jnp.sum(p, axis=1, keepdims=True)
                ctxh = lax.dot_general(p.astype(jnp.bfloat16), vt_ref[h],
                                       (((1,), (0,)), ((), ())),
                                       preferred_element_type=jnp.float32)
                ctx_buf[pl.ds(r0, CHUNK), h, :] = ctxh
                ml_buf[pl.ds(r0, CHUNK), h:h + 1] = m
                ml_buf[pl.ds(r0, CHUNK), HQ + h:HQ + h + 1] = lsum

        def merge_recv(r0, slot):
            rs = pl.ds(r0, CHUNK)
            m_loc = ml_buf[rs, 0:HQ]
            l_loc = ml_buf[rs, HQ:2 * HQ]
            m_in = recv_ml[slot, :, 0:HQ]
            l_in = recv_ml[slot, :, HQ:2 * HQ]
            m_new = jnp.maximum(m_loc, m_in)
            a = jnp.exp2(m_loc - m_new)
            b = jnp.exp2(m_in - m_new)
            ml_buf[rs, 0:HQ] = m_new
            ml_buf[rs, HQ:2 * HQ] = a * l_loc + b * l_in
            ctx_buf[rs, :, :] = (a[:, :, None] * ctx_buf[rs, :, :]
                                 + b[:, :, None] * recv_ctx[slot])

        pending = []
        for t in range(N_DEV):
            c = lax.rem(my - t + N_DEV, N_DEV)
            r0 = c * CHUNK
            compute_chunk(r0)
            if t > 0:
                pending[2 * (t - 1)].wait_recv()
                pending[2 * (t - 1) + 1].wait_recv()
                merge_recv(r0, t - 1)
            if t < N_DEV - 1:
                rc = pltpu.make_async_remote_copy(
                    src_ref=ctx_buf.at[pl.ds(r0, CHUNK)],
                    dst_ref=recv_ctx.at[t],
                    send_sem=rs_send.at[t], recv_sem=rs_recv.at[t],
                    device_id=(right,), device_id_type=pl.DeviceIdType.MESH)
                rm = pltpu.make_async_remote_copy(
                    src_ref=ml_buf.at[pl.ds(r0, CHUNK)],
                    dst_ref=recv_ml.at[t],
                    send_sem=ml_send.at[t], recv_sem=ml_recv.at[t],
                    device_id=(right,), device_id_type=pl.DeviceIdType.MESH)
                rc.start()
                rm.start()
                pending.extend([rc, rm])

        own = lax.rem(my + 1, N_DEV)
        r0o = own * CHUNK
        rso = pl.ds(r0o, CHUNK)
        l_fin = ml_buf[rso, HQ:2 * HQ]
        acc = jnp.zeros((CHUNK, DM), jnp.float32)
        for h in range(HQ):
            ch = (ctx_buf[rso, h, :] / l_fin[:, h:h + 1]).astype(jnp.bfloat16)
            acc = acc + lax.dot_general(
                ch, wo_ref[h * DH:(h + 1) * DH, :], (((1,), (0,)), ((), ())),
                preferred_element_type=jnp.float32)
        out_ref[0, rso, :] = acc

        N_R, N_L = 3, N_DEV - 1 - 3
        prev_r = prev_l = None
        for u in range(max(N_R, N_L)):
            if u < N_R:
                g = lax.rem(my + 1 - u + N_DEV, N_DEV)
                rg = pl.ds(g * CHUNK, CHUNK)
                if prev_r is not None:
                    prev_r.wait_recv()
                agr = pltpu.make_async_remote_copy(
                    src_ref=out_ref.at[0, rg], dst_ref=out_ref.at[0, rg],
                    send_sem=ag_send.at[u], recv_sem=ag_recv.at[u],
                    device_id=(right,), device_id_type=pl.DeviceIdType.MESH)
                agr.start()
                pending.append(agr)
                prev_r = agr
            if u < N_L:
                g = lax.rem(my + 1 + u, N_DEV)
                rg = pl.ds(g * CHUNK, CHUNK)
                if prev_l is not None:
                    prev_l.wait_recv()
                agl = pltpu.make_async_remote_copy(
                    src_ref=out_ref.at[0, rg], dst_ref=out_ref.at[0, rg],
                    send_sem=agl_send.at[u], recv_sem=agl_recv.at[u],
                    device_id=(left,), device_id_type=pl.DeviceIdType.MESH)
                agl.start()
                pending.append(agl)
                prev_l = agl
        prev_r.wait_recv()
        prev_l.wait_recv()

        for r in pending:
            r.wait_send()

    call = pl.pallas_call(
        body,
        out_shape=jax.ShapeDtypeStruct((1, SQ, DM), jnp.float32),
        in_specs=[pl.BlockSpec(memory_space=pltpu.VMEM)] * 5,
        out_specs=pl.BlockSpec(memory_space=pltpu.VMEM),
        scratch_shapes=[
            pltpu.VMEM((SQ, DM), jnp.bfloat16),
            pltpu.VMEM((SQ, HQ, DH), jnp.float32),
            pltpu.VMEM((SQ, 2 * HQ), jnp.float32),
            pltpu.VMEM((N_DEV - 1, CHUNK, HQ, DH), jnp.float32),
            pltpu.VMEM((N_DEV - 1, CHUNK, 2 * HQ), jnp.float32),
            pltpu.SemaphoreType.DMA((N_DEV - 1,)),
            pltpu.SemaphoreType.DMA((N_DEV - 1,)),
            pltpu.SemaphoreType.DMA((N_DEV - 1,)),
            pltpu.SemaphoreType.DMA((N_DEV - 1,)),
            pltpu.SemaphoreType.DMA((N_DEV - 1,)),
            pltpu.SemaphoreType.DMA((N_DEV - 1,)),
            pltpu.SemaphoreType.DMA((N_DEV - 1,)),
            pltpu.SemaphoreType.DMA((N_DEV - 1,)),
        ],
        compiler_params=pltpu.CompilerParams(
            collective_id=0, vmem_limit_bytes=63 * 1024 * 1024),
    )
    kt = jnp.transpose(K_ext[0].astype(jnp.bfloat16), (1, 0, 2))
    vt = jnp.transpose(V_ext[0].astype(jnp.bfloat16), (1, 0, 2))
    return call(x.astype(jnp.bfloat16), Wq.astype(jnp.bfloat16),
                kt, vt, Wo.astype(jnp.bfloat16))


# device time: 195459 ns/iter; 1.1132x vs baseline; 1.1132x over previous
import jax
import jax.numpy as jnp
from jax import lax
from jax.experimental import pallas as pl
from jax.experimental.pallas import tpu as pltpu

N_DEV = 8
SQ = 2048
SKV = 2048
HQ = 8
DH = 128
DM = 1024
BLK = 64
CHUNK = SQ // N_DEV
LOG2E = 1.4426950408889634
SCALE = 0.08838834764831843 * LOG2E
NEG = -1.5e9


def kernel(x, Wq, K_ext, V_ext, Wo):
    def body(x_ref, wq_ref, kt_ref, vt_ref, wo_ref, out_ref,
             q_buf, ctx_buf, ml_buf, recv_ctx, recv_ml, out_bf,
             rs_send, rs_recv, ml_send, ml_recv, ag_send, ag_recv,
             agl_send, agl_recv):
        my = lax.axis_index("i")
        left = lax.rem(my + N_DEV - 1, N_DEV)
        right = lax.rem(my + 1, N_DEV)

        barrier = pltpu.get_barrier_semaphore()
        for nbr in (left, right):
            pl.semaphore_signal(barrier, inc=1, device_id=(nbr,),
                                device_id_type=pl.DeviceIdType.MESH)
        pl.semaphore_wait(barrier, 2)

        def qproj_step(c, carry):
            r0 = c * CHUNK
            q_buf[pl.ds(r0, CHUNK), :] = (lax.dot_general(
                x_ref[0, pl.ds(r0, CHUNK), :], wq_ref[...],
                (((1,), (0,)), ((), ())),
                preferred_element_type=jnp.float32) * SCALE).astype(jnp.bfloat16)
            return carry

        lax.fori_loop(0, N_DEV, qproj_step, 0)

        kb_off = my * (SKV // BLK)

        def compute_chunk(r0):
            qb = (lax.broadcasted_iota(jnp.int32, (CHUNK, SKV), 0) + r0) // BLK
            kb = lax.broadcasted_iota(jnp.int32, (CHUNK, SKV), 1) // BLK + kb_off
            active = (qb == kb) | (kb == 0) | (lax.rem(qb + kb, 3) == 0)
            bias = jnp.where(active, 0.0, NEG).astype(jnp.float32)
            for h in range(HQ):
                qh = q_buf[pl.ds(r0, CHUNK), h * DH:(h + 1) * DH]
                s = lax.dot_general(qh, kt_ref[h], (((1,), (1,)), ((), ())),
                                    preferred_element_type=jnp.float32) + bias
                m = jnp.max(s, axis=1, keepdims=True)
                p = jnp.exp2(s - m)
                lsum = jnp.sum(p, axis=1, keepdims=True)
                ctxh = lax.dot_general(p.astype(jnp.bfloat16), vt_ref[h],
                                       (((1,), (0,)), ((), ())),
                                       preferred_element_type=jnp.float32)
                ctx_buf[pl.ds(r0, CHUNK), h, :] = ctxh
                ml_buf[pl.ds(r0, CHUNK), h:h + 1] = m
                ml_buf[pl.ds(r0, CHUNK), HQ + h:HQ + h + 1] = lsum

        def merge_recv(r0, slot):
            rs = pl.ds(r0, CHUNK)
            m_loc = ml_buf[rs, 0:HQ]
            l_loc = ml_buf[rs, HQ:2 * HQ]
            m_in = recv_ml[slot, :, 0:HQ]
            l_in = recv_ml[slot, :, HQ:2 * HQ]
            m_new = jnp.maximum(m_loc, m_in)
            a = jnp.exp2(m_loc - m_new)
            b = jnp.exp2(m_in - m_new)
            ml_buf[rs, 0:HQ] = m_new
            ml_buf[rs, HQ:2 * HQ] = a * l_loc + b * l_in
            ctx_buf[rs, :, :] = (a[:, :, None] * ctx_buf[rs, :, :]
                                 + b[:, :, None] * recv_ctx[slot])

        pending = []
        for t in range(N_DEV):
            c = lax.rem(my - t + N_DEV, N_DEV)
            r0 = c * CHUNK
            compute_chunk(r0)
            if t > 0:
                pending[2 * (t - 1)].wait_recv()
                pending[2 * (t - 1) + 1].wait_recv()
                merge_recv(r0, t - 1)
            if t < N_DEV - 1:
                rc = pltpu.make_async_remote_copy(
                    src_ref=ctx_buf.at[pl.ds(r0, CHUNK)],
                    dst_ref=recv_ctx.at[t],
                    send_sem=rs_send.at[t], recv_sem=rs_recv.at[t],
                    device_id=(right,), device_id_type=pl.DeviceIdType.MESH)
                rm = pltpu.make_async_remote_copy(
                    src_ref=ml_buf.at[pl.ds(r0, CHUNK)],
                    dst_ref=recv_ml.at[t],
                    send_sem=ml_send.at[t], recv_sem=ml_recv.at[t],
                    device_id=(right,), device_id_type=pl.DeviceIdType.MESH)
                rc.start()
                rm.start()
                pending.extend([rc, rm])

        own = lax.rem(my + 1, N_DEV)
        r0o = own * CHUNK
        rso = pl.ds(r0o, CHUNK)
        l_fin = ml_buf[rso, HQ:2 * HQ]
        acc = jnp.zeros((CHUNK, DM), jnp.float32)
        for h in range(HQ):
            ch = (ctx_buf[rso, h, :] / l_fin[:, h:h + 1]).astype(jnp.bfloat16)
            acc = acc + lax.dot_general(
                ch, wo_ref[h * DH:(h + 1) * DH, :], (((1,), (0,)), ((), ())),
                preferred_element_type=jnp.float32)
        out_ref[0, rso, :] = acc
        out_bf[rso, :] = acc.astype(jnp.bfloat16)

        N_R, N_L = 3, N_DEV - 1 - 3
        prev_r = prev_l = None
        for u in range(max(N_R, N_L)):
            if u < N_R:
                g = lax.rem(my + 1 - u + N_DEV, N_DEV)
                rg = pl.ds(g * CHUNK, CHUNK)
                if prev_r is not None:
                    prev_r.wait_recv()
                agr = pltpu.make_async_remote_copy(
                    src_ref=out_bf.at[rg], dst_ref=out_bf.at[rg],
                    send_sem=ag_send.at[u], recv_sem=ag_recv.at[u],
                    device_id=(right,), device_id_type=pl.DeviceIdType.MESH)
                agr.start()
                pending.append(agr)
                prev_r = agr
            if u < N_L:
                g = lax.rem(my + 1 + u, N_DEV)
                rg = pl.ds(g * CHUNK, CHUNK)
                if prev_l is not None:
                    prev_l.wait_recv()
                agl = pltpu.make_async_remote_copy(
                    src_ref=out_bf.at[rg], dst_ref=out_bf.at[rg],
                    send_sem=agl_send.at[u], recv_sem=agl_recv.at[u],
                    device_id=(left,), device_id_type=pl.DeviceIdType.MESH)
                agl.start()
                pending.append(agl)
                prev_l = agl
        prev_r.wait_recv()
        prev_l.wait_recv()

        for d in (0, 1, 2):
            rg = pl.ds(lax.rem(my - d + N_DEV, N_DEV) * CHUNK, CHUNK)
            out_ref[0, rg, :] = out_bf[rg, :].astype(jnp.float32)
        for d in (2, 3, 4, 5):
            rg = pl.ds(lax.rem(my + d, N_DEV) * CHUNK, CHUNK)
            out_ref[0, rg, :] = out_bf[rg, :].astype(jnp.float32)

        for r in pending:
            r.wait_send()

    call = pl.pallas_call(
        body,
        out_shape=jax.ShapeDtypeStruct((1, SQ, DM), jnp.float32),
        in_specs=[pl.BlockSpec(memory_space=pltpu.VMEM)] * 5,
        out_specs=pl.BlockSpec(memory_space=pltpu.VMEM),
        scratch_shapes=[
            pltpu.VMEM((SQ, DM), jnp.bfloat16),
            pltpu.VMEM((SQ, HQ, DH), jnp.float32),
            pltpu.VMEM((SQ, 2 * HQ), jnp.float32),
            pltpu.VMEM((N_DEV - 1, CHUNK, HQ, DH), jnp.float32),
            pltpu.VMEM((N_DEV - 1, CHUNK, 2 * HQ), jnp.float32),
            pltpu.VMEM((SQ, DM), jnp.bfloat16),
            pltpu.SemaphoreType.DMA((N_DEV - 1,)),
            pltpu.SemaphoreType.DMA((N_DEV - 1,)),
            pltpu.SemaphoreType.DMA((N_DEV - 1,)),
            pltpu.SemaphoreType.DMA((N_DEV - 1,)),
            pltpu.SemaphoreType.DMA((N_DEV - 1,)),
            pltpu.SemaphoreType.DMA((N_DEV - 1,)),
            pltpu.SemaphoreType.DMA((N_DEV - 1,)),
            pltpu.SemaphoreType.DMA((N_DEV - 1,)),
        ],
        compiler_params=pltpu.CompilerParams(
            collective_id=0, vmem_limit_bytes=63 * 1024 * 1024),
    )
    kt = jnp.transpose(K_ext[0].astype(jnp.bfloat16), (1, 0, 2))
    vt = jnp.transpose(V_ext[0].astype(jnp.bfloat16), (1, 0, 2))
    return call(x.astype(jnp.bfloat16), Wq.astype(jnp.bfloat16),
                kt, vt, Wo.astype(jnp.bfloat16))
